# baseline (speedup 1.0000x reference)
import functools

import jax
import jax.numpy as jnp
from jax import lax
from jax.experimental import pallas as pl
from jax.experimental.pallas import tpu as pltpu
from jax.experimental.pallas import tpu_sc as plsc

D_MODEL = 768
NUM_EXPERTS = 8
EXPERT_HIDDEN = 2048
S = 2048

T = 256
NT = (2 * S + NUM_EXPERTS * T) // T
R = NT * T

TPW = S // 32


def _router_kernel(x_ref, wr_ref, br_ref,
                   e1_ref, e2_ref, w1_ref, w2_ref, rcinit_ref, sc_ref):
    x = x_ref[...]
    logits = jnp.dot(x, wr_ref[...],
                     preferred_element_type=jnp.float32) + br_ref[...]
    l1 = jnp.max(logits, axis=-1, keepdims=True)
    e1 = jnp.argmax(logits, axis=-1).astype(jnp.int32)
    cols = lax.broadcasted_iota(jnp.int32, logits.shape, 1)
    masked = jnp.where(cols == e1[:, None], -jnp.inf, logits)
    e2 = jnp.argmax(masked, axis=-1).astype(jnp.int32)
    l2 = jnp.max(masked, axis=-1, keepdims=True)
    s1 = 1.0 / (1.0 + jnp.exp(l2 - l1))
    e1_ref[...] = e1
    e2_ref[...] = e2
    w1_ref[...] = s1[:, 0]
    w2_ref[...] = 1.0 - s1[:, 0]

    io16 = lax.broadcasted_iota(jnp.int32, (S, 16), 1)
    cnt = ((io16 == e1[:, None]).astype(jnp.int32)
           + (io16 == e2[:, None]).astype(jnp.int32))
    total = jnp.sum(cnt, axis=0)
    pc = ((total + T - 1) // T) * T
    rows = lax.broadcasted_iota(jnp.int32, (16, 16), 0)
    colsq = lax.broadcasted_iota(jnp.int32, (16, 16), 1)
    seg = jnp.sum(jnp.where(colsq < rows, pc[None, :], 0), axis=1)
    seg_end = seg + pc
    ti = lax.broadcasted_iota(jnp.int32, (NT, 16), 0) * T
    te = jnp.clip(jnp.sum((ti >= seg_end[None, :]).astype(jnp.int32), axis=1),
                  0, NUM_EXPERTS - 1)

    used = (total > 0).astype(jnp.int32)
    ru = jnp.sum(pc)
    tstart = lax.broadcasted_iota(jnp.int32, (NT, 1), 0)[:, 0] * T
    act = (tstart < ru).astype(jnp.int32)
    tprev = (tstart - T)
    teprev = jnp.clip(jnp.sum((tprev[:, None] >= seg_end[None, :])
                              .astype(jnp.int32), axis=1), 0, NUM_EXPERTS - 1)
    ii = lax.broadcasted_iota(jnp.int32, (NT, 1), 0)[:, 0]
    first = jnp.where((ii == 0) | (te != teprev), 1, 0) * act
    ordn = jnp.sum(used[None, :] * (seg[None, :] <= ti).astype(jnp.int32),
                   axis=1)
    slot = jnp.where(ordn > 0, (ordn - 1) % 3, 0)
    eidx = lax.broadcasted_iota(jnp.int32, (NT, 16), 1)
    cand = jnp.where((used[None, :] > 0) & (seg[None, :] > ti), eidx, 99)
    pf = jnp.min(cand, axis=1)
    cand2 = jnp.where(eidx > pf[:, None], cand, 99)
    pf2 = jnp.min(cand2, axis=1)
    pfgo = ((pf < NUM_EXPERTS) & (ii == 0)).astype(jnp.int32) * act
    pf2go = ((pf2 < NUM_EXPERTS) & (first > 0)).astype(jnp.int32) * act
    sc_ref[0, :] = te
    sc_ref[1, :] = first
    sc_ref[2, :] = slot
    sc_ref[3, :] = jnp.clip(pf, 0, NUM_EXPERTS - 1)
    sc_ref[4, :] = pfgo
    sc_ref[5, :] = act
    sc_ref[6, :] = jnp.clip(pf2, 0, NUM_EXPERTS - 1)
    sc_ref[7, :] = pf2go

    ww = lax.broadcasted_iota(jnp.int32, (32, S), 0)
    tw = lax.broadcasted_iota(jnp.int32, (32, S), 1) // TPW
    sel = (ww == tw).astype(jnp.float32)
    wc = jnp.dot(sel, cnt.astype(jnp.float32),
                 preferred_element_type=jnp.float32)
    wr_ = lax.broadcasted_iota(jnp.int32, (32, 32), 0)
    wc_ = lax.broadcasted_iota(jnp.int32, (32, 32), 1)
    trilw = (wc_ < wr_).astype(jnp.float32)
    pw = jnp.dot(trilw, wc, preferred_element_type=jnp.float32)
    rcinit_ref[...] = seg[None, :] + pw.astype(jnp.int32)


def _router():
    return pl.pallas_call(
        _router_kernel,
        grid=(1,),
        in_specs=[
            pl.BlockSpec((S, D_MODEL), lambda i: (0, 0)),
            pl.BlockSpec((D_MODEL, NUM_EXPERTS), lambda i: (0, 0)),
            pl.BlockSpec((NUM_EXPERTS,), lambda i: (0,)),
        ],
        out_specs=[
            pl.BlockSpec((S,), lambda i: (0,)),
            pl.BlockSpec((S,), lambda i: (0,)),
            pl.BlockSpec((S,), lambda i: (0,)),
            pl.BlockSpec((S,), lambda i: (0,)),
            pl.BlockSpec((32, 16), lambda i: (0, 0)),
            pl.BlockSpec((8, NT), lambda i: (0, 0)),
        ],
        out_shape=[
            jax.ShapeDtypeStruct((S,), jnp.int32),
            jax.ShapeDtypeStruct((S,), jnp.int32),
            jax.ShapeDtypeStruct((S,), jnp.float32),
            jax.ShapeDtypeStruct((S,), jnp.float32),
            jax.ShapeDtypeStruct((32, 16), jnp.int32),
            jax.ShapeDtypeStruct((8, NT), jnp.int32),
        ],
    )


_SC_MESH = plsc.VectorSubcoreMesh(core_axis_name="c", subcore_axis_name="s")

_GDN = lax.GatherDimensionNumbers(
    offset_dims=(), collapsed_slice_dims=(0,), start_index_map=(0,))


def _vgather(vec, idx):
    return lax.gather(vec, idx[:, None], _GDN, (1,),
                      mode=lax.GatherScatterMode.PROMISE_IN_BOUNDS)


@functools.partial(
    pl.kernel,
    out_type=(
        jax.ShapeDtypeStruct((R, D_MODEL), jnp.float32),
        jax.ShapeDtypeStruct((S,), jnp.int32),
        jax.ShapeDtypeStruct((S,), jnp.int32),
    ),
    mesh=_SC_MESH,
    scratch_types=[
        pltpu.VMEM((TPW,), jnp.int32),
        pltpu.VMEM((TPW,), jnp.int32),
        pltpu.VMEM((16,), jnp.int32),
        pltpu.VMEM((TPW, D_MODEL), jnp.float32),
        pltpu.VMEM((TPW,), jnp.int32),
        pltpu.VMEM((TPW,), jnp.int32),
        pltpu.SemaphoreType.DMA,
    ],
)
def _dispatch(x_hbm, e1_hbm, e2_hbm, rcinit_hbm,
              xs_hbm, pos1_hbm, pos2_hbm,
              e1_v, e2_v, rc_v, rows_v, d1_v, d2_v, sem):
    cid = lax.axis_index("c")
    sid = lax.axis_index("s")
    wid = sid * 2 + cid
    base = wid * TPW

    pltpu.sync_copy(e1_hbm.at[pl.ds(base, TPW)], e1_v)
    pltpu.sync_copy(e2_hbm.at[pl.ds(base, TPW)], e2_v)
    pltpu.sync_copy(rcinit_hbm.at[wid], rc_v)
    pltpu.sync_copy(x_hbm.at[pl.ds(base, TPW)], rows_v)

    lane = lax.iota(jnp.int32, 16)
    rc = rc_v[...]

    for ev, dv in ((e1_v, d1_v), (e2_v, d2_v)):
        for cidx in range(TPW // 16):
            v = ev[pl.ds(cidx * 16, 16)]
            start = _vgather(rc, v)
            rank = jnp.zeros((16,), jnp.int32)
            cnt = jnp.zeros((16,), jnp.int32)
            for lp in range(16):
                es = _vgather(v, jnp.full((16,), lp, jnp.int32))
                eqv = 1 - jnp.minimum(jnp.abs(v - es), 1)
                gt = jnp.minimum(jnp.maximum(lane - lp, 0), 1)
                rank = rank + eqv * gt
                cnt = cnt + (1 - jnp.minimum(jnp.abs(lane - es), 1))
            dst = start + rank
            dst = jnp.minimum(jnp.maximum(dst, 0), R - 1)
            dv[pl.ds(cidx * 16, 16)] = dst
            rc = rc + cnt

    pltpu.sync_copy(d1_v, pos1_hbm.at[pl.ds(base, TPW)])
    pltpu.sync_copy(d2_v, pos2_hbm.at[pl.ds(base, TPW)])
    c1 = pltpu.async_copy(rows_v, xs_hbm.at[d1_v], sem)
    c2 = pltpu.async_copy(rows_v, xs_hbm.at[d2_v], sem)
    c1.wait()
    c2.wait()


def _ffn_kernel(sc_ref, xs_ref, b1_ref, b2_ref, w1_hbm, w2_hbm, out_ref,
                w1buf, w2buf, sem1, sem2):
    i = pl.program_id(0)
    cur = sc_ref[0, i]
    fst = sc_ref[1, i]
    slot = sc_ref[2, i]
    pf = sc_ref[3, i]
    pfgo = sc_ref[4, i]
    act = sc_ref[5, i]
    pf2 = sc_ref[6, i]
    pf2go = sc_ref[7, i]

    @pl.when((i == 0) & (act == 1))
    def _():
        pltpu.make_async_copy(w1_hbm.at[cur], w1buf.at[slot],
                              sem1.at[slot]).start()
        pltpu.make_async_copy(w2_hbm.at[cur], w2buf.at[slot],
                              sem2.at[slot]).start()

    @pl.when(pfgo == 1)
    def _():
        s1 = lax.rem(slot + 1, 3)
        pltpu.make_async_copy(w1_hbm.at[pf], w1buf.at[s1],
                              sem1.at[s1]).start()
        pltpu.make_async_copy(w2_hbm.at[pf], w2buf.at[s1],
                              sem2.at[s1]).start()

    @pl.when(pf2go == 1)
    def _():
        s2 = lax.rem(slot + 2, 3)
        pltpu.make_async_copy(w1_hbm.at[pf2], w1buf.at[s2],
                              sem1.at[s2]).start()
        pltpu.make_async_copy(w2_hbm.at[pf2], w2buf.at[s2],
                              sem2.at[s2]).start()

    @pl.when(fst == 1)
    def _():
        pltpu.make_async_copy(w1_hbm.at[cur], w1buf.at[slot],
                              sem1.at[slot]).wait()
        pltpu.make_async_copy(w2_hbm.at[cur], w2buf.at[slot],
                              sem2.at[slot]).wait()

    @pl.when(act == 1)
    def _():
        xt = xs_ref[...]
        hid = jnp.maximum(
            jnp.dot(xt, w1buf[slot], preferred_element_type=jnp.float32)
            + b1_ref[cur, 0], 0.0)
        out_ref[...] = (jnp.dot(hid, w2buf[slot],
                                preferred_element_type=jnp.float32)
                        + b2_ref[cur, 0])


def _ffn(sc, xs, W1, b1r, W2, b2r):
    grid_spec = pltpu.PrefetchScalarGridSpec(
        num_scalar_prefetch=1,
        grid=(NT,),
        in_specs=[
            pl.BlockSpec((T, D_MODEL), lambda i, sc: (i, 0)),
            pl.BlockSpec((NUM_EXPERTS, 1, EXPERT_HIDDEN),
                         lambda i, sc: (0, 0, 0)),
            pl.BlockSpec((NUM_EXPERTS, 1, D_MODEL), lambda i, sc: (0, 0, 0)),
            pl.BlockSpec(memory_space=pl.ANY),
            pl.BlockSpec(memory_space=pl.ANY),
        ],
        out_specs=pl.BlockSpec((T, D_MODEL), lambda i, sc: (i, 0)),
        scratch_shapes=[
            pltpu.VMEM((3, D_MODEL, EXPERT_HIDDEN), jnp.float32),
            pltpu.VMEM((3, EXPERT_HIDDEN, D_MODEL), jnp.float32),
            pltpu.SemaphoreType.DMA((3,)),
            pltpu.SemaphoreType.DMA((3,)),
        ],
    )
    return pl.pallas_call(
        _ffn_kernel,
        grid_spec=grid_spec,
        out_shape=jax.ShapeDtypeStruct((R, D_MODEL), jnp.float32),
    )(sc, xs, b1r, b2r, W1, W2)


_CHW = 32


@functools.partial(
    pl.kernel,
    out_type=jax.ShapeDtypeStruct((S, D_MODEL), jnp.float32),
    mesh=_SC_MESH,
    scratch_types=[
        pltpu.VMEM((_CHW,), jnp.int32),
        pltpu.VMEM((_CHW,), jnp.int32),
        pltpu.VMEM((_CHW,), jnp.float32),
        pltpu.VMEM((_CHW,), jnp.float32),
        pltpu.VMEM((_CHW, D_MODEL), jnp.float32),
        pltpu.VMEM((_CHW, D_MODEL), jnp.float32),
        pltpu.SemaphoreType.DMA,
    ],
)
def _combine(ys_hbm, pos1_hbm, pos2_hbm, w1_hbm, w2_hbm, out_hbm,
             iA, iB, wA, wB, rA, rB, sem):
    cid = lax.axis_index("c")
    sid = lax.axis_index("s")
    wid = sid * 2 + cid
    base = wid * TPW

    for hh in range(TPW // _CHW):
        o = base + hh * _CHW
        pltpu.sync_copy(pos1_hbm.at[pl.ds(o, _CHW)], iA)
        pltpu.sync_copy(pos2_hbm.at[pl.ds(o, _CHW)], iB)
        pltpu.sync_copy(w1_hbm.at[pl.ds(o, _CHW)], wA)
        pltpu.sync_copy(w2_hbm.at[pl.ds(o, _CHW)], wB)
        cA = pltpu.async_copy(ys_hbm.at[iA], rA, sem)
        cB = pltpu.async_copy(ys_hbm.at[iB], rB, sem)
        cA.wait()
        cB.wait()
        for cj in range(_CHW // 16):
            wva = wA[pl.ds(cj * 16, 16)]
            wvb = wB[pl.ds(cj * 16, 16)]

            def jbody(lj, carry, wva=wva, wvb=wvb, cj=cj):
                j = cj * 16 + lj
                sel = jnp.full((16,), lj, jnp.int32)
                sa = _vgather(wva, sel)
                sb = _vgather(wvb, sel)
                for k in range(D_MODEL // 16):
                    sl = pl.ds(k * 16, 16)
                    rA[j, sl] = sa * rA[j, sl] + sb * rB[j, sl]
                return carry
            lax.fori_loop(0, 16, jbody, 0)
        pltpu.sync_copy(rA, out_hbm.at[pl.ds(o, _CHW)])


def kernel(x, Wr, br, W1, b1, W2, b2):
    x2 = x.reshape(S, D_MODEL)
    e1, e2, w1, w2, rcinit, sc = _router()(x2, Wr, br)
    xs, pos1, pos2 = _dispatch(x2, e1, e2, rcinit)
    ys = _ffn(sc, xs, W1,
              b1.reshape(NUM_EXPERTS, 1, EXPERT_HIDDEN),
              W2, b2.reshape(NUM_EXPERTS, 1, D_MODEL))
    out = _combine(ys, pos1, pos2, w1, w2)
    return out.reshape(x.shape)


# speedup vs baseline: 1.0186x; 1.0186x over previous
import functools

import jax
import jax.numpy as jnp
from jax import lax
from jax.experimental import pallas as pl
from jax.experimental.pallas import tpu as pltpu
from jax.experimental.pallas import tpu_sc as plsc

D_MODEL = 768
NUM_EXPERTS = 8
EXPERT_HIDDEN = 2048
S = 2048

T = 256
NT = (2 * S + NUM_EXPERTS * T) // T
R = NT * T

TPW = S // 32


def _router_kernel(x_ref, wr_ref, br_ref,
                   e1_ref, e2_ref, w1_ref, w2_ref, rcinit_ref, sc_ref):
    x = x_ref[...]
    logits = jnp.dot(x, wr_ref[...],
                     preferred_element_type=jnp.float32) + br_ref[...]
    l1 = jnp.max(logits, axis=-1, keepdims=True)
    e1 = jnp.argmax(logits, axis=-1).astype(jnp.int32)
    cols = lax.broadcasted_iota(jnp.int32, logits.shape, 1)
    masked = jnp.where(cols == e1[:, None], -jnp.inf, logits)
    e2 = jnp.argmax(masked, axis=-1).astype(jnp.int32)
    l2 = jnp.max(masked, axis=-1, keepdims=True)
    s1 = 1.0 / (1.0 + jnp.exp(l2 - l1))
    e1_ref[...] = e1
    e2_ref[...] = e2
    w1_ref[...] = s1[:, 0]
    w2_ref[...] = 1.0 - s1[:, 0]

    io16 = lax.broadcasted_iota(jnp.int32, (S, 16), 1)
    cnt = ((io16 == e1[:, None]).astype(jnp.int32)
           + (io16 == e2[:, None]).astype(jnp.int32))
    total = jnp.sum(cnt, axis=0)
    pc = ((total + T - 1) // T) * T
    rows = lax.broadcasted_iota(jnp.int32, (16, 16), 0)
    colsq = lax.broadcasted_iota(jnp.int32, (16, 16), 1)
    seg = jnp.sum(jnp.where(colsq < rows, pc[None, :], 0), axis=1)
    seg_end = seg + pc
    ti = lax.broadcasted_iota(jnp.int32, (NT, 16), 0) * T
    te = jnp.clip(jnp.sum((ti >= seg_end[None, :]).astype(jnp.int32), axis=1),
                  0, NUM_EXPERTS - 1)

    used = (total > 0).astype(jnp.int32)
    ru = jnp.sum(pc)
    tstart = lax.broadcasted_iota(jnp.int32, (NT, 1), 0)[:, 0] * T
    act = (tstart < ru).astype(jnp.int32)
    tprev = (tstart - T)
    teprev = jnp.clip(jnp.sum((tprev[:, None] >= seg_end[None, :])
                              .astype(jnp.int32), axis=1), 0, NUM_EXPERTS - 1)
    ii = lax.broadcasted_iota(jnp.int32, (NT, 1), 0)[:, 0]
    first = jnp.where((ii == 0) | (te != teprev), 1, 0) * act
    ordn = jnp.sum(used[None, :] * (seg[None, :] <= ti).astype(jnp.int32),
                   axis=1)
    slot = jnp.where(ordn > 0, (ordn - 1) % 2, 0)
    eidx = lax.broadcasted_iota(jnp.int32, (NT, 16), 1)
    cand = jnp.where((used[None, :] > 0) & (seg[None, :] > ti), eidx, 99)
    pf = jnp.min(cand, axis=1)
    cand2 = jnp.where(eidx > pf[:, None], cand, 99)
    pf2 = jnp.min(cand2, axis=1)
    pfgo = ((pf < NUM_EXPERTS) & (first > 0)).astype(jnp.int32) * act
    pf2go = ((pf2 < NUM_EXPERTS) & (first > 0)).astype(jnp.int32) * act
    sc_ref[0, :] = te
    sc_ref[1, :] = first
    sc_ref[2, :] = slot
    sc_ref[3, :] = jnp.clip(pf, 0, NUM_EXPERTS - 1)
    sc_ref[4, :] = pfgo
    sc_ref[5, :] = act
    sc_ref[6, :] = jnp.clip(pf2, 0, NUM_EXPERTS - 1)
    sc_ref[7, :] = pf2go

    ww = lax.broadcasted_iota(jnp.int32, (32, S), 0)
    tw = lax.broadcasted_iota(jnp.int32, (32, S), 1) // TPW
    sel = (ww == tw).astype(jnp.float32)
    wc = jnp.dot(sel, cnt.astype(jnp.float32),
                 preferred_element_type=jnp.float32)
    wr_ = lax.broadcasted_iota(jnp.int32, (32, 32), 0)
    wc_ = lax.broadcasted_iota(jnp.int32, (32, 32), 1)
    trilw = (wc_ < wr_).astype(jnp.float32)
    pw = jnp.dot(trilw, wc, preferred_element_type=jnp.float32)
    rcinit_ref[...] = seg[None, :] + pw.astype(jnp.int32)


def _router():
    return pl.pallas_call(
        _router_kernel,
        grid=(1,),
        in_specs=[
            pl.BlockSpec((S, D_MODEL), lambda i: (0, 0)),
            pl.BlockSpec((D_MODEL, NUM_EXPERTS), lambda i: (0, 0)),
            pl.BlockSpec((NUM_EXPERTS,), lambda i: (0,)),
        ],
        out_specs=[
            pl.BlockSpec((S,), lambda i: (0,)),
            pl.BlockSpec((S,), lambda i: (0,)),
            pl.BlockSpec((S,), lambda i: (0,)),
            pl.BlockSpec((S,), lambda i: (0,)),
            pl.BlockSpec((32, 16), lambda i: (0, 0)),
            pl.BlockSpec((8, NT), lambda i: (0, 0)),
        ],
        out_shape=[
            jax.ShapeDtypeStruct((S,), jnp.int32),
            jax.ShapeDtypeStruct((S,), jnp.int32),
            jax.ShapeDtypeStruct((S,), jnp.float32),
            jax.ShapeDtypeStruct((S,), jnp.float32),
            jax.ShapeDtypeStruct((32, 16), jnp.int32),
            jax.ShapeDtypeStruct((8, NT), jnp.int32),
        ],
    )


_SC_MESH = plsc.VectorSubcoreMesh(core_axis_name="c", subcore_axis_name="s")

_GDN = lax.GatherDimensionNumbers(
    offset_dims=(), collapsed_slice_dims=(0,), start_index_map=(0,))


def _vgather(vec, idx):
    return lax.gather(vec, idx[:, None], _GDN, (1,),
                      mode=lax.GatherScatterMode.PROMISE_IN_BOUNDS)


@functools.partial(
    pl.kernel,
    out_type=(
        jax.ShapeDtypeStruct((R, D_MODEL), jnp.float32),
        jax.ShapeDtypeStruct((S,), jnp.int32),
        jax.ShapeDtypeStruct((S,), jnp.int32),
    ),
    mesh=_SC_MESH,
    scratch_types=[
        pltpu.VMEM((TPW,), jnp.int32),
        pltpu.VMEM((TPW,), jnp.int32),
        pltpu.VMEM((16,), jnp.int32),
        pltpu.VMEM((TPW, D_MODEL), jnp.float32),
        pltpu.VMEM((TPW,), jnp.int32),
        pltpu.VMEM((TPW,), jnp.int32),
        pltpu.SemaphoreType.DMA,
    ],
)
def _dispatch(x_hbm, e1_hbm, e2_hbm, rcinit_hbm,
              xs_hbm, pos1_hbm, pos2_hbm,
              e1_v, e2_v, rc_v, rows_v, d1_v, d2_v, sem):
    cid = lax.axis_index("c")
    sid = lax.axis_index("s")
    wid = sid * 2 + cid
    base = wid * TPW

    pltpu.sync_copy(e1_hbm.at[pl.ds(base, TPW)], e1_v)
    pltpu.sync_copy(e2_hbm.at[pl.ds(base, TPW)], e2_v)
    pltpu.sync_copy(rcinit_hbm.at[wid], rc_v)
    pltpu.sync_copy(x_hbm.at[pl.ds(base, TPW)], rows_v)

    lane = lax.iota(jnp.int32, 16)
    rc = rc_v[...]

    for ev, dv in ((e1_v, d1_v), (e2_v, d2_v)):
        for cidx in range(TPW // 16):
            v = ev[pl.ds(cidx * 16, 16)]
            start = _vgather(rc, v)
            rank = jnp.zeros((16,), jnp.int32)
            cnt = jnp.zeros((16,), jnp.int32)
            for lp in range(16):
                es = _vgather(v, jnp.full((16,), lp, jnp.int32))
                eqv = 1 - jnp.minimum(jnp.abs(v - es), 1)
                gt = jnp.minimum(jnp.maximum(lane - lp, 0), 1)
                rank = rank + eqv * gt
                cnt = cnt + (1 - jnp.minimum(jnp.abs(lane - es), 1))
            dst = start + rank
            dst = jnp.minimum(jnp.maximum(dst, 0), R - 1)
            dv[pl.ds(cidx * 16, 16)] = dst
            rc = rc + cnt

    pltpu.sync_copy(d1_v, pos1_hbm.at[pl.ds(base, TPW)])
    pltpu.sync_copy(d2_v, pos2_hbm.at[pl.ds(base, TPW)])
    c1 = pltpu.async_copy(rows_v, xs_hbm.at[d1_v], sem)
    c2 = pltpu.async_copy(rows_v, xs_hbm.at[d2_v], sem)
    c1.wait()
    c2.wait()


def _ffn_kernel(sc_ref, xs_ref, b1_ref, b2_ref, w1_hbm, w2_hbm, out_ref,
                w1buf, w2buf, sem1, sem2):
    i = pl.program_id(0)
    cur = sc_ref[0, i]
    fst = sc_ref[1, i]
    slot = sc_ref[2, i]
    pf = sc_ref[3, i]
    pfgo = sc_ref[4, i]
    act = sc_ref[5, i]
    pf2 = sc_ref[6, i]
    pf2go = sc_ref[7, i]

    @pl.when((i == 0) & (act == 1))
    def _():
        pltpu.make_async_copy(w1_hbm.at[cur], w1buf.at[slot],
                              sem1.at[slot]).start()
        pltpu.make_async_copy(w2_hbm.at[cur], w2buf.at[slot],
                              sem2.at[slot]).start()

    @pl.when(pfgo == 1)
    def _():
        pltpu.make_async_copy(w1_hbm.at[pf], w1buf.at[1 - slot],
                              sem1.at[1 - slot]).start()
        pltpu.make_async_copy(w2_hbm.at[pf], w2buf.at[1 - slot],
                              sem2.at[1 - slot]).start()

    @pl.when(fst == 1)
    def _():
        pltpu.make_async_copy(w1_hbm.at[cur], w1buf.at[slot],
                              sem1.at[slot]).wait()
        pltpu.make_async_copy(w2_hbm.at[cur], w2buf.at[slot],
                              sem2.at[slot]).wait()

    @pl.when(act == 1)
    def _():
        xt = xs_ref[...]
        hid = jnp.maximum(
            jnp.dot(xt, w1buf[slot], preferred_element_type=jnp.float32)
            + b1_ref[cur, 0], 0.0)
        out_ref[...] = (jnp.dot(hid, w2buf[slot],
                                preferred_element_type=jnp.float32)
                        + b2_ref[cur, 0])


def _ffn(sc, xs, W1, b1r, W2, b2r):
    grid_spec = pltpu.PrefetchScalarGridSpec(
        num_scalar_prefetch=1,
        grid=(NT,),
        in_specs=[
            pl.BlockSpec((T, D_MODEL), lambda i, sc: (i, 0)),
            pl.BlockSpec((NUM_EXPERTS, 1, EXPERT_HIDDEN),
                         lambda i, sc: (0, 0, 0)),
            pl.BlockSpec((NUM_EXPERTS, 1, D_MODEL), lambda i, sc: (0, 0, 0)),
            pl.BlockSpec(memory_space=pl.ANY),
            pl.BlockSpec(memory_space=pl.ANY),
        ],
        out_specs=pl.BlockSpec((T, D_MODEL), lambda i, sc: (i, 0)),
        scratch_shapes=[
            pltpu.VMEM((2, D_MODEL, EXPERT_HIDDEN), jnp.float32),
            pltpu.VMEM((2, EXPERT_HIDDEN, D_MODEL), jnp.float32),
            pltpu.SemaphoreType.DMA((2,)),
            pltpu.SemaphoreType.DMA((2,)),
        ],
    )
    return pl.pallas_call(
        _ffn_kernel,
        grid_spec=grid_spec,
        out_shape=jax.ShapeDtypeStruct((R, D_MODEL), jnp.float32),
    )(sc, xs, b1r, b2r, W1, W2)


_CHW = 32


@functools.partial(
    pl.kernel,
    out_type=jax.ShapeDtypeStruct((S, D_MODEL), jnp.float32),
    mesh=_SC_MESH,
    scratch_types=[
        pltpu.VMEM((_CHW,), jnp.int32),
        pltpu.VMEM((_CHW,), jnp.int32),
        pltpu.VMEM((_CHW,), jnp.float32),
        pltpu.VMEM((_CHW,), jnp.float32),
        pltpu.VMEM((_CHW, D_MODEL), jnp.float32),
        pltpu.VMEM((_CHW, D_MODEL), jnp.float32),
        pltpu.SemaphoreType.DMA,
    ],
)
def _combine(ys_hbm, pos1_hbm, pos2_hbm, w1_hbm, w2_hbm, out_hbm,
             iA, iB, wA, wB, rA, rB, sem):
    cid = lax.axis_index("c")
    sid = lax.axis_index("s")
    wid = sid * 2 + cid
    base = wid * TPW

    for hh in range(TPW // _CHW):
        o = base + hh * _CHW
        pltpu.sync_copy(pos1_hbm.at[pl.ds(o, _CHW)], iA)
        pltpu.sync_copy(pos2_hbm.at[pl.ds(o, _CHW)], iB)
        pltpu.sync_copy(w1_hbm.at[pl.ds(o, _CHW)], wA)
        pltpu.sync_copy(w2_hbm.at[pl.ds(o, _CHW)], wB)
        cA = pltpu.async_copy(ys_hbm.at[iA], rA, sem)
        cB = pltpu.async_copy(ys_hbm.at[iB], rB, sem)
        cA.wait()
        cB.wait()
        for cj in range(_CHW // 16):
            wva = wA[pl.ds(cj * 16, 16)]
            wvb = wB[pl.ds(cj * 16, 16)]

            def jbody(lj, carry, wva=wva, wvb=wvb, cj=cj):
                j = cj * 16 + lj
                sel = jnp.full((16,), lj, jnp.int32)
                sa = _vgather(wva, sel)
                sb = _vgather(wvb, sel)
                for k in range(D_MODEL // 16):
                    sl = pl.ds(k * 16, 16)
                    rA[j, sl] = sa * rA[j, sl] + sb * rB[j, sl]
                return carry
            lax.fori_loop(0, 16, jbody, 0)
        pltpu.sync_copy(rA, out_hbm.at[pl.ds(o, _CHW)])


def kernel(x, Wr, br, W1, b1, W2, b2):
    x2 = x.reshape(S, D_MODEL)
    e1, e2, w1, w2, rcinit, sc = _router()(x2, Wr, br)
    xs, pos1, pos2 = _dispatch(x2, e1, e2, rcinit)
    ys = _ffn(sc, xs, W1,
              b1.reshape(NUM_EXPERTS, 1, EXPERT_HIDDEN),
              W2, b2.reshape(NUM_EXPERTS, 1, D_MODEL))
    out = _combine(ys, pos1, pos2, w1, w2)
    return out.reshape(x.shape)


# speedup vs baseline: 1.0558x; 1.0365x over previous
import functools

import jax
import jax.numpy as jnp
from jax import lax
from jax.experimental import pallas as pl
from jax.experimental.pallas import tpu as pltpu
from jax.experimental.pallas import tpu_sc as plsc

D_MODEL = 768
NUM_EXPERTS = 8
EXPERT_HIDDEN = 2048
S = 2048

T = 256
NT = (2 * S + NUM_EXPERTS * T) // T
R = NT * T

TPW = S // 32


def _router_kernel(x_ref, wr_ref, br_ref,
                   e1_ref, e2_ref, w1_ref, w2_ref, rcinit_ref, sc_ref):
    x = x_ref[...]
    logits = jnp.dot(x, wr_ref[...],
                     preferred_element_type=jnp.float32) + br_ref[...]
    l1 = jnp.max(logits, axis=-1, keepdims=True)
    e1 = jnp.argmax(logits, axis=-1).astype(jnp.int32)
    cols = lax.broadcasted_iota(jnp.int32, logits.shape, 1)
    masked = jnp.where(cols == e1[:, None], -jnp.inf, logits)
    e2 = jnp.argmax(masked, axis=-1).astype(jnp.int32)
    l2 = jnp.max(masked, axis=-1, keepdims=True)
    s1 = 1.0 / (1.0 + jnp.exp(l2 - l1))
    e1_ref[...] = e1
    e2_ref[...] = e2
    w1_ref[...] = s1[:, 0]
    w2_ref[...] = 1.0 - s1[:, 0]

    io16 = lax.broadcasted_iota(jnp.int32, (S, 16), 1)
    cnt = ((io16 == e1[:, None]).astype(jnp.int32)
           + (io16 == e2[:, None]).astype(jnp.int32))
    total = jnp.sum(cnt, axis=0)
    pc = ((total + T - 1) // T) * T
    rows = lax.broadcasted_iota(jnp.int32, (16, 16), 0)
    colsq = lax.broadcasted_iota(jnp.int32, (16, 16), 1)
    seg = jnp.sum(jnp.where(colsq < rows, pc[None, :], 0), axis=1)
    seg_end = seg + pc
    ti = lax.broadcasted_iota(jnp.int32, (NT, 16), 0) * T
    te = jnp.clip(jnp.sum((ti >= seg_end[None, :]).astype(jnp.int32), axis=1),
                  0, NUM_EXPERTS - 1)

    used = (total > 0).astype(jnp.int32)
    ru = jnp.sum(pc)
    tstart = lax.broadcasted_iota(jnp.int32, (NT, 1), 0)[:, 0] * T
    act = (tstart < ru).astype(jnp.int32)
    tprev = (tstart - T)
    teprev = jnp.clip(jnp.sum((tprev[:, None] >= seg_end[None, :])
                              .astype(jnp.int32), axis=1), 0, NUM_EXPERTS - 1)
    ii = lax.broadcasted_iota(jnp.int32, (NT, 1), 0)[:, 0]
    first = jnp.where((ii == 0) | (te != teprev), 1, 0) * act
    ordn = jnp.sum(used[None, :] * (seg[None, :] <= ti).astype(jnp.int32),
                   axis=1)
    slot = jnp.where(ordn > 0, (ordn - 1) % 2, 0)
    eidx = lax.broadcasted_iota(jnp.int32, (NT, 16), 1)
    cand = jnp.where((used[None, :] > 0) & (seg[None, :] > ti), eidx, 99)
    pf = jnp.min(cand, axis=1)
    cand2 = jnp.where(eidx > pf[:, None], cand, 99)
    pf2 = jnp.min(cand2, axis=1)
    pfgo = ((pf < NUM_EXPERTS) & (first > 0)).astype(jnp.int32) * act
    pf2go = ((pf2 < NUM_EXPERTS) & (first > 0)).astype(jnp.int32) * act
    sc_ref[0, :] = te
    sc_ref[1, :] = first
    sc_ref[2, :] = slot
    sc_ref[3, :] = jnp.clip(pf, 0, NUM_EXPERTS - 1)
    sc_ref[4, :] = pfgo
    sc_ref[5, :] = act
    sc_ref[6, :] = jnp.clip(pf2, 0, NUM_EXPERTS - 1)
    sc_ref[7, :] = pf2go

    ww = lax.broadcasted_iota(jnp.int32, (32, S), 0)
    tw = lax.broadcasted_iota(jnp.int32, (32, S), 1) // TPW
    sel = (ww == tw).astype(jnp.float32)
    wc = jnp.dot(sel, cnt.astype(jnp.float32),
                 preferred_element_type=jnp.float32)
    wr_ = lax.broadcasted_iota(jnp.int32, (32, 32), 0)
    wc_ = lax.broadcasted_iota(jnp.int32, (32, 32), 1)
    trilw = (wc_ < wr_).astype(jnp.float32)
    pw = jnp.dot(trilw, wc, preferred_element_type=jnp.float32)
    rcinit_ref[...] = seg[None, :] + pw.astype(jnp.int32)


def _router():
    return pl.pallas_call(
        _router_kernel,
        grid=(1,),
        in_specs=[
            pl.BlockSpec((S, D_MODEL), lambda i: (0, 0)),
            pl.BlockSpec((D_MODEL, NUM_EXPERTS), lambda i: (0, 0)),
            pl.BlockSpec((NUM_EXPERTS,), lambda i: (0,)),
        ],
        out_specs=[
            pl.BlockSpec((S,), lambda i: (0,)),
            pl.BlockSpec((S,), lambda i: (0,)),
            pl.BlockSpec((S,), lambda i: (0,)),
            pl.BlockSpec((S,), lambda i: (0,)),
            pl.BlockSpec((32, 16), lambda i: (0, 0)),
            pl.BlockSpec((8, NT), lambda i: (0, 0)),
        ],
        out_shape=[
            jax.ShapeDtypeStruct((S,), jnp.int32),
            jax.ShapeDtypeStruct((S,), jnp.int32),
            jax.ShapeDtypeStruct((S,), jnp.float32),
            jax.ShapeDtypeStruct((S,), jnp.float32),
            jax.ShapeDtypeStruct((32, 16), jnp.int32),
            jax.ShapeDtypeStruct((8, NT), jnp.int32),
        ],
    )


_SC_MESH = plsc.VectorSubcoreMesh(core_axis_name="c", subcore_axis_name="s")

_GDN = lax.GatherDimensionNumbers(
    offset_dims=(), collapsed_slice_dims=(0,), start_index_map=(0,))


def _vgather(vec, idx):
    return lax.gather(vec, idx[:, None], _GDN, (1,),
                      mode=lax.GatherScatterMode.PROMISE_IN_BOUNDS)


@functools.partial(
    pl.kernel,
    out_type=(
        jax.ShapeDtypeStruct((R, D_MODEL), jnp.float32),
        jax.ShapeDtypeStruct((S,), jnp.int32),
        jax.ShapeDtypeStruct((S,), jnp.int32),
    ),
    mesh=_SC_MESH,
    scratch_types=[
        pltpu.VMEM((TPW,), jnp.int32),
        pltpu.VMEM((TPW,), jnp.int32),
        pltpu.VMEM((16,), jnp.int32),
        pltpu.VMEM((TPW, D_MODEL), jnp.float32),
        pltpu.VMEM((TPW,), jnp.int32),
        pltpu.VMEM((TPW,), jnp.int32),
        pltpu.SemaphoreType.DMA,
    ],
)
def _dispatch(x_hbm, e1_hbm, e2_hbm, rcinit_hbm,
              xs_hbm, pos1_hbm, pos2_hbm,
              e1_v, e2_v, rc_v, rows_v, d1_v, d2_v, sem):
    cid = lax.axis_index("c")
    sid = lax.axis_index("s")
    wid = sid * 2 + cid
    base = wid * TPW

    l1 = pltpu.async_copy(e1_hbm.at[pl.ds(base, TPW)], e1_v, sem)
    l2 = pltpu.async_copy(e2_hbm.at[pl.ds(base, TPW)], e2_v, sem)
    l3 = pltpu.async_copy(rcinit_hbm.at[wid], rc_v, sem)
    l4 = pltpu.async_copy(x_hbm.at[pl.ds(base, TPW)], rows_v, sem)
    l1.wait()
    l2.wait()
    l3.wait()
    l4.wait()

    lane = lax.iota(jnp.int32, 16)
    rc = rc_v[...]

    for ev, dv in ((e1_v, d1_v), (e2_v, d2_v)):
        for cidx in range(TPW // 16):
            v = ev[pl.ds(cidx * 16, 16)]
            start = _vgather(rc, v)
            rank = jnp.zeros((16,), jnp.int32)
            cnt = jnp.zeros((16,), jnp.int32)
            for lp in range(16):
                es = _vgather(v, jnp.full((16,), lp, jnp.int32))
                eqv = 1 - jnp.minimum(jnp.abs(v - es), 1)
                gt = jnp.minimum(jnp.maximum(lane - lp, 0), 1)
                rank = rank + eqv * gt
                cnt = cnt + (1 - jnp.minimum(jnp.abs(lane - es), 1))
            dst = start + rank
            dst = jnp.minimum(jnp.maximum(dst, 0), R - 1)
            dv[pl.ds(cidx * 16, 16)] = dst
            rc = rc + cnt

    pltpu.sync_copy(d1_v, pos1_hbm.at[pl.ds(base, TPW)])
    pltpu.sync_copy(d2_v, pos2_hbm.at[pl.ds(base, TPW)])
    c1 = pltpu.async_copy(rows_v, xs_hbm.at[d1_v], sem)
    c2 = pltpu.async_copy(rows_v, xs_hbm.at[d2_v], sem)
    c1.wait()
    c2.wait()


def _ffn_kernel(sc_ref, xs_ref, b1_ref, b2_ref, w1_hbm, w2_hbm, out_ref,
                w1buf, w2buf, sem1, sem2):
    i = pl.program_id(0)
    cur = sc_ref[0, i]
    fst = sc_ref[1, i]
    slot = sc_ref[2, i]
    pf = sc_ref[3, i]
    pfgo = sc_ref[4, i]
    act = sc_ref[5, i]
    pf2 = sc_ref[6, i]
    pf2go = sc_ref[7, i]

    @pl.when((i == 0) & (act == 1))
    def _():
        pltpu.make_async_copy(w1_hbm.at[cur], w1buf.at[slot],
                              sem1.at[slot]).start()
        pltpu.make_async_copy(w2_hbm.at[cur], w2buf.at[slot],
                              sem2.at[slot]).start()

    @pl.when(pfgo == 1)
    def _():
        pltpu.make_async_copy(w1_hbm.at[pf], w1buf.at[1 - slot],
                              sem1.at[1 - slot]).start()
        pltpu.make_async_copy(w2_hbm.at[pf], w2buf.at[1 - slot],
                              sem2.at[1 - slot]).start()

    @pl.when(fst == 1)
    def _():
        pltpu.make_async_copy(w1_hbm.at[cur], w1buf.at[slot],
                              sem1.at[slot]).wait()
        pltpu.make_async_copy(w2_hbm.at[cur], w2buf.at[slot],
                              sem2.at[slot]).wait()

    @pl.when(act == 1)
    def _():
        xt = xs_ref[...]
        hid = jnp.maximum(
            jnp.dot(xt, w1buf[slot], preferred_element_type=jnp.float32)
            + b1_ref[cur, 0], 0.0)
        out_ref[...] = (jnp.dot(hid, w2buf[slot],
                                preferred_element_type=jnp.float32)
                        + b2_ref[cur, 0])


def _ffn(sc, xs, W1, b1r, W2, b2r):
    grid_spec = pltpu.PrefetchScalarGridSpec(
        num_scalar_prefetch=1,
        grid=(NT,),
        in_specs=[
            pl.BlockSpec((T, D_MODEL), lambda i, sc: (i, 0)),
            pl.BlockSpec((NUM_EXPERTS, 1, EXPERT_HIDDEN),
                         lambda i, sc: (0, 0, 0)),
            pl.BlockSpec((NUM_EXPERTS, 1, D_MODEL), lambda i, sc: (0, 0, 0)),
            pl.BlockSpec(memory_space=pl.ANY),
            pl.BlockSpec(memory_space=pl.ANY),
        ],
        out_specs=pl.BlockSpec((T, D_MODEL), lambda i, sc: (i, 0)),
        scratch_shapes=[
            pltpu.VMEM((2, D_MODEL, EXPERT_HIDDEN), jnp.float32),
            pltpu.VMEM((2, EXPERT_HIDDEN, D_MODEL), jnp.float32),
            pltpu.SemaphoreType.DMA((2,)),
            pltpu.SemaphoreType.DMA((2,)),
        ],
    )
    return pl.pallas_call(
        _ffn_kernel,
        grid_spec=grid_spec,
        out_shape=jax.ShapeDtypeStruct((R, D_MODEL), jnp.float32),
    )(sc, xs, b1r, b2r, W1, W2)


_CHW = 32


@functools.partial(
    pl.kernel,
    out_type=jax.ShapeDtypeStruct((S, D_MODEL), jnp.float32),
    mesh=_SC_MESH,
    scratch_types=[
        pltpu.VMEM((_CHW,), jnp.int32),
        pltpu.VMEM((_CHW,), jnp.int32),
        pltpu.VMEM((_CHW,), jnp.float32),
        pltpu.VMEM((_CHW,), jnp.float32),
        pltpu.VMEM((_CHW, D_MODEL), jnp.float32),
        pltpu.VMEM((_CHW, D_MODEL), jnp.float32),
        pltpu.SemaphoreType.DMA,
    ],
)
def _combine(ys_hbm, pos1_hbm, pos2_hbm, w1_hbm, w2_hbm, out_hbm,
             iA, iB, wA, wB, rA, rB, sem):
    cid = lax.axis_index("c")
    sid = lax.axis_index("s")
    wid = sid * 2 + cid
    base = wid * TPW

    for hh in range(TPW // _CHW):
        o = base + hh * _CHW
        c1 = pltpu.async_copy(pos1_hbm.at[pl.ds(o, _CHW)], iA, sem)
        c2 = pltpu.async_copy(pos2_hbm.at[pl.ds(o, _CHW)], iB, sem)
        c3 = pltpu.async_copy(w1_hbm.at[pl.ds(o, _CHW)], wA, sem)
        c4 = pltpu.async_copy(w2_hbm.at[pl.ds(o, _CHW)], wB, sem)
        c1.wait()
        c2.wait()
        c3.wait()
        c4.wait()
        cA = pltpu.async_copy(ys_hbm.at[iA], rA, sem)
        cB = pltpu.async_copy(ys_hbm.at[iB], rB, sem)
        cA.wait()
        cB.wait()
        for cj in range(_CHW // 16):
            wva = wA[pl.ds(cj * 16, 16)]
            wvb = wB[pl.ds(cj * 16, 16)]

            def jbody(lj, carry, wva=wva, wvb=wvb, cj=cj):
                j = cj * 16 + lj
                sel = jnp.full((16,), lj, jnp.int32)
                sa = _vgather(wva, sel)
                sb = _vgather(wvb, sel)
                for k in range(D_MODEL // 16):
                    sl = pl.ds(k * 16, 16)
                    rA[j, sl] = sa * rA[j, sl] + sb * rB[j, sl]
                return carry
            lax.fori_loop(0, 16, jbody, 0)
        pltpu.sync_copy(rA, out_hbm.at[pl.ds(o, _CHW)])


def kernel(x, Wr, br, W1, b1, W2, b2):
    x2 = x.reshape(S, D_MODEL)
    e1, e2, w1, w2, rcinit, sc = _router()(x2, Wr, br)
    xs, pos1, pos2 = _dispatch(x2, e1, e2, rcinit)
    ys = _ffn(sc, xs, W1,
              b1.reshape(NUM_EXPERTS, 1, EXPERT_HIDDEN),
              W2, b2.reshape(NUM_EXPERTS, 1, D_MODEL))
    out = _combine(ys, pos1, pos2, w1, w2)
    return out.reshape(x.shape)
